# Initial kernel scaffold; baseline (speedup 1.0000x reference)
import jax
import jax.numpy as jnp
from jax.experimental import pallas as pl


def kernel(x, W_r, b_r, W_up, b_up, W_down, b_down):
    raise NotImplementedError("write your pallas kernel here")


# speedup vs baseline: 1.8908x; 1.8908x over previous
import functools

import jax
import jax.numpy as jnp
from jax import lax
from jax.experimental import pallas as pl
from jax.experimental.pallas import tpu as pltpu
from jax.experimental.pallas import tpu_sc as plsc

_B, _SEQ, _H = 2, 2048, 768
_E = 64
_GROUPS = 16
_GSIZE = _E // _GROUPS
_FFN = 2048
_N = _B * _SEQ
_K = _N // _E

_NC, _NS = 2, 16
_NW = _NC * _NS
_ROWS_PER_W = _N // _NW
_ROWS_PER_TILE = _N // _NS
_HH = _H // _NC
_CHUNK = 64
_NCHUNK = _ROWS_PER_TILE // _CHUNK


def _router_body(x_ref, wr_ref, br_ref, g_ref, i_ref):
    h = jnp.dot(x_ref[...], wr_ref[...], preferred_element_type=jnp.float32)
    h = h + br_ref[...]
    m = jnp.max(h, axis=1, keepdims=True)
    ex = jnp.exp(h - m)
    s = jnp.sum(ex, axis=1, keepdims=True)
    sm = ex / s
    iota0 = lax.broadcasted_iota(jnp.int32, sm.shape, 0)

    def step(j, run):
        cmax = jnp.max(run, axis=0, keepdims=True)
        hit = run == cmax
        isel = jnp.min(jnp.where(hit, iota0, _N), axis=0, keepdims=True)
        g_ref[pl.ds(j, 1), :] = cmax
        i_ref[pl.ds(j, 1), :] = isel
        return jnp.where(iota0 == isel, -1.0, run)

    lax.fori_loop(0, _K, step, sm)


def _router(xf, W_r, b_r):
    return pl.pallas_call(
        _router_body,
        out_shape=(
            jax.ShapeDtypeStruct((_K, _E), jnp.float32),
            jax.ShapeDtypeStruct((_K, _E), jnp.int32),
        ),
    )(xf, W_r, b_r.reshape(1, _E))


def _gather(xf, idx_w):
    mesh = plsc.VectorSubcoreMesh(core_axis_name="c", subcore_axis_name="s")

    @functools.partial(
        pl.kernel,
        mesh=mesh,
        out_type=jax.ShapeDtypeStruct((_N, _H), jnp.float32),
        scratch_types=[
            pltpu.VMEM((_ROWS_PER_W,), jnp.int32),
            pltpu.VMEM((_ROWS_PER_W, _H), jnp.float32),
            pltpu.SemaphoreType.DMA,
        ],
    )
    def k(x_hbm, idx_hbm, out_hbm, idx_v, rows_v, sem):
        wid = lax.axis_index("s") * _NC + lax.axis_index("c")
        pltpu.sync_copy(idx_hbm.at[wid], idx_v)
        pltpu.async_copy(x_hbm.at[idx_v], rows_v, sem).wait()
        pltpu.sync_copy(rows_v, out_hbm.at[pl.ds(wid * _ROWS_PER_W, _ROWS_PER_W)])

    return k(xf, idx_w)


def _ffn_body(t_ref, wu_ref, bu_ref, wd_ref, bd_ref, g_ref, o_ref):
    up = jnp.dot(t_ref[0], wu_ref[0], preferred_element_type=jnp.float32)
    up = up + bu_ref[0]
    hid = up * (1.0 / (1.0 + jnp.exp(-up)))
    out = jnp.dot(hid, wd_ref[0], preferred_element_type=jnp.float32)
    out = out + bd_ref[0]
    o_ref[0] = (out * g_ref[0]).astype(jnp.bfloat16)


def _ffn(tokens, W_up, b_up, W_down, b_down, G_ek1):
    return pl.pallas_call(
        _ffn_body,
        grid=(_E,),
        in_specs=[
            pl.BlockSpec((1, _K, _H), lambda e: (e, 0, 0)),
            pl.BlockSpec((1, _H, _FFN), lambda e: (e // _GSIZE, 0, 0)),
            pl.BlockSpec((1, 1, _FFN), lambda e: (e // _GSIZE, 0, 0)),
            pl.BlockSpec((1, _FFN, _H), lambda e: (e, 0, 0)),
            pl.BlockSpec((1, 1, _H), lambda e: (e, 0, 0)),
            pl.BlockSpec((1, _K, 1), lambda e: (e, 0, 0)),
        ],
        out_specs=pl.BlockSpec((1, _K, _H), lambda e: (e, 0, 0)),
        out_shape=jax.ShapeDtypeStruct((_E, _K, _H), jnp.bfloat16),
    )(tokens, W_up, b_up, W_down, b_down, G_ek1)


_TB = 256


def _scatter_body(idx_ref, w_ref, y_ref):
    t = pl.program_id(0)
    tok = t * _TB + lax.broadcasted_iota(jnp.int32, (_TB, 1), 0)
    oh = (tok == idx_ref[...]).astype(jnp.bfloat16)
    y_ref[...] = jnp.dot(oh, w_ref[...], preferred_element_type=jnp.float32)


def _scatter_add(weighted_bf16, idx_row):
    return pl.pallas_call(
        _scatter_body,
        grid=(_N // _TB,),
        in_specs=[
            pl.BlockSpec((1, _N), lambda t: (0, 0)),
            pl.BlockSpec((_N, _H), lambda t: (0, 0)),
        ],
        out_specs=pl.BlockSpec((_TB, _H), lambda t: (t, 0)),
        out_shape=jax.ShapeDtypeStruct((_N, _H), jnp.float32),
    )(idx_row, weighted_bf16)


def kernel(x, W_r, b_r, W_up, b_up, W_down, b_down):
    xf = x.reshape(_N, _H)
    g_ke, i_ke = _router(xf, W_r, b_r)
    idx_flat = i_ke.T.reshape(-1)
    tokens = _gather(xf, idx_flat.reshape(_NW, _ROWS_PER_W))
    weighted = _ffn(
        tokens.reshape(_E, _K, _H),
        W_up, b_up.reshape(_GROUPS, 1, _FFN),
        W_down, b_down.reshape(_E, 1, _H),
        g_ke.T.reshape(_E, _K, 1),
    )
    y = _scatter_add(weighted.reshape(_N, _H), idx_flat.reshape(1, _N))
    return y.reshape(_B, _SEQ, _H)


# speedup vs baseline: 1.9796x; 1.0470x over previous
import functools

import jax
import jax.numpy as jnp
from jax import lax
from jax.experimental import pallas as pl
from jax.experimental.pallas import tpu as pltpu
from jax.experimental.pallas import tpu_sc as plsc

_B, _SEQ, _H = 2, 2048, 768
_E = 64
_GROUPS = 16
_GSIZE = _E // _GROUPS
_FFN = 2048
_N = _B * _SEQ
_K = _N // _E

_NC, _NS = 2, 16
_NW = _NC * _NS
_ROWS_PER_W = _N // _NW
_ROWS_PER_TILE = _N // _NS
_HH = _H // _NC
_CHUNK = 64
_NCHUNK = _ROWS_PER_TILE // _CHUNK


def _router_body(x_ref, wr_ref, br_ref, g_ref, i_ref):
    h = jnp.dot(x_ref[...], wr_ref[...], preferred_element_type=jnp.float32)
    h = h + br_ref[...]
    m = jnp.max(h, axis=1, keepdims=True)
    ex = jnp.exp(h - m)
    s = jnp.sum(ex, axis=1, keepdims=True)
    sm = ex / s
    iota0 = lax.broadcasted_iota(jnp.int32, sm.shape, 0)

    def step(j, run):
        cmax = jnp.max(run, axis=0, keepdims=True)
        hit = run == cmax
        isel = jnp.min(jnp.where(hit, iota0, _N), axis=0, keepdims=True)
        g_ref[pl.ds(j, 1), :] = cmax
        i_ref[pl.ds(j, 1), :] = isel
        return jnp.where(iota0 == isel, -1.0, run)

    lax.fori_loop(0, _K, step, sm)


def _router(xf, W_r, b_r):
    return pl.pallas_call(
        _router_body,
        out_shape=(
            jax.ShapeDtypeStruct((_K, _E), jnp.float32),
            jax.ShapeDtypeStruct((_K, _E), jnp.int32),
        ),
    )(xf, W_r, b_r.reshape(1, _E))


def _gather(xf, idx_w):
    mesh = plsc.VectorSubcoreMesh(core_axis_name="c", subcore_axis_name="s")

    @functools.partial(
        pl.kernel,
        mesh=mesh,
        out_type=jax.ShapeDtypeStruct((_N, _H), jnp.float32),
        scratch_types=[
            pltpu.VMEM((_ROWS_PER_W,), jnp.int32),
            pltpu.VMEM((_ROWS_PER_W, _H), jnp.float32),
            pltpu.SemaphoreType.DMA,
        ],
    )
    def k(x_hbm, idx_hbm, out_hbm, idx_v, rows_v, sem):
        wid = lax.axis_index("s") * _NC + lax.axis_index("c")
        pltpu.sync_copy(idx_hbm.at[wid], idx_v)
        pltpu.async_copy(x_hbm.at[idx_v], rows_v, sem).wait()
        pltpu.sync_copy(rows_v, out_hbm.at[pl.ds(wid * _ROWS_PER_W, _ROWS_PER_W)])

    return k(xf, idx_w)


def _ffn_body(idx_sref, t_ref, wu_ref, bu_ref, wd_ref, bd_ref, g_ref, y_ref):
    e = pl.program_id(0)

    @pl.when(e == 0)
    def _init():
        y_ref[...] = jnp.zeros((_N, _H), jnp.float32)

    t = t_ref[0].astype(jnp.bfloat16)
    wu = wu_ref[0].astype(jnp.bfloat16)
    up = jnp.dot(t, wu, preferred_element_type=jnp.float32)
    up = up + bu_ref[0]
    hid = up * (1.0 / (1.0 + jnp.exp(-up)))
    wd = wd_ref[0].astype(jnp.bfloat16)
    out = jnp.dot(hid.astype(jnp.bfloat16), wd, preferred_element_type=jnp.float32)
    out = out + bd_ref[0]
    weighted = out * g_ref[0]
    for i in range(_K):
        tok = idx_sref[e, i]
        y_ref[pl.ds(tok, 1), :] += weighted[i:i + 1, :]


def _ffn_scatter(idx_ek, tokens, W_up, b_up, W_down, b_down, G_ek1):
    return pl.pallas_call(
        _ffn_body,
        grid_spec=pltpu.PrefetchScalarGridSpec(
            num_scalar_prefetch=1,
            grid=(_E,),
            in_specs=[
                pl.BlockSpec((1, _K, _H), lambda e, s: (e, 0, 0)),
                pl.BlockSpec((1, _H, _FFN), lambda e, s: (e // _GSIZE, 0, 0)),
                pl.BlockSpec((1, 1, _FFN), lambda e, s: (e // _GSIZE, 0, 0)),
                pl.BlockSpec((1, _FFN, _H), lambda e, s: (e, 0, 0)),
                pl.BlockSpec((1, 1, _H), lambda e, s: (e, 0, 0)),
                pl.BlockSpec((1, _K, 1), lambda e, s: (e, 0, 0)),
            ],
            out_specs=pl.BlockSpec((_N, _H), lambda e, s: (0, 0)),
        ),
        out_shape=jax.ShapeDtypeStruct((_N, _H), jnp.float32),
    )(idx_ek, tokens, W_up, b_up, W_down, b_down, G_ek1)


def kernel(x, W_r, b_r, W_up, b_up, W_down, b_down):
    xf = x.reshape(_N, _H)
    g_ke, i_ke = _router(xf, W_r, b_r)
    idx_ek = i_ke.T
    tokens = _gather(xf, idx_ek.reshape(_NW, _ROWS_PER_W))
    y = _ffn_scatter(
        idx_ek,
        tokens.reshape(_E, _K, _H),
        W_up, b_up.reshape(_GROUPS, 1, _FFN),
        W_down, b_down.reshape(_E, 1, _H),
        g_ke.T.reshape(_E, _K, 1),
    )
    return y.reshape(_B, _SEQ, _H)


# speedup vs baseline: 2.1251x; 1.0735x over previous
import functools

import jax
import jax.numpy as jnp
from jax import lax
from jax.experimental import pallas as pl
from jax.experimental.pallas import tpu as pltpu
from jax.experimental.pallas import tpu_sc as plsc

_B, _SEQ, _H = 2, 2048, 768
_E = 64
_GROUPS = 16
_GSIZE = _E // _GROUPS
_FFN = 2048
_N = _B * _SEQ
_K = _N // _E

_NC, _NS = 2, 16
_NW = _NC * _NS
_ROWS_PER_W = _N // _NW
_ROWS_PER_TILE = _N // _NS
_HH = _H // _NC
_CHUNK = 64
_NCHUNK = _ROWS_PER_TILE // _CHUNK


def _router_body(x_ref, wr_ref, br_ref, g_ref, i_ref):
    hT = lax.dot_general(
        wr_ref[...], x_ref[...], (((0,), (1,)), ((), ())),
        preferred_element_type=jnp.float32,
    )
    hT = hT + br_ref[...]
    m = jnp.max(hT, axis=0, keepdims=True)
    ex = jnp.exp(hT - m)
    s = jnp.sum(ex, axis=0, keepdims=True)
    sm = ex / s
    iota1 = lax.broadcasted_iota(jnp.int32, sm.shape, 1)
    iota_k = lax.broadcasted_iota(jnp.int32, (_E, _K), 1)

    def step(j, carry):
        run, g_acc, i_acc = carry
        cmax = jnp.max(run, axis=1, keepdims=True)
        isel = jnp.min(jnp.where(run == cmax, iota1, _N), axis=1, keepdims=True)
        g_acc = jnp.where(iota_k == j, cmax, g_acc)
        i_acc = jnp.where(iota_k == j, isel, i_acc)
        run = jnp.where(iota1 == isel, -1.0, run)
        return run, g_acc, i_acc

    _, g_acc, i_acc = lax.fori_loop(
        0, _K, step,
        (sm, jnp.zeros((_E, _K), jnp.float32), jnp.zeros((_E, _K), jnp.int32)),
    )
    g_ref[...] = g_acc
    i_ref[...] = i_acc


def _router(xf, W_r, b_r):
    return pl.pallas_call(
        _router_body,
        out_shape=(
            jax.ShapeDtypeStruct((_E, _K), jnp.float32),
            jax.ShapeDtypeStruct((_E, _K), jnp.int32),
        ),
    )(xf, W_r, b_r.reshape(_E, 1))


def _gather(xf, idx_w):
    mesh = plsc.VectorSubcoreMesh(core_axis_name="c", subcore_axis_name="s")

    @functools.partial(
        pl.kernel,
        mesh=mesh,
        out_type=jax.ShapeDtypeStruct((_N, _H), jnp.float32),
        scratch_types=[
            pltpu.VMEM((_ROWS_PER_W,), jnp.int32),
            pltpu.VMEM((_ROWS_PER_W, _H), jnp.float32),
            pltpu.SemaphoreType.DMA,
        ],
    )
    def k(x_hbm, idx_hbm, out_hbm, idx_v, rows_v, sem):
        wid = lax.axis_index("s") * _NC + lax.axis_index("c")
        pltpu.sync_copy(idx_hbm.at[wid], idx_v)
        pltpu.async_copy(x_hbm.at[idx_v], rows_v, sem).wait()
        pltpu.sync_copy(rows_v, out_hbm.at[pl.ds(wid * _ROWS_PER_W, _ROWS_PER_W)])

    return k(xf, idx_w)


def _ffn_body(idx_sref, t_ref, wu_ref, bu_ref, wd_ref, bd_ref, g_ref, y_ref):
    e = pl.program_id(0)

    @pl.when(e == 0)
    def _init():
        y_ref[...] = jnp.zeros((_N, _H), jnp.float32)

    t = t_ref[0].astype(jnp.bfloat16)
    wu = wu_ref[0].astype(jnp.bfloat16)
    up = jnp.dot(t, wu, preferred_element_type=jnp.float32)
    up = up + bu_ref[0]
    hid = up * (1.0 / (1.0 + jnp.exp(-up)))
    wd = wd_ref[0].astype(jnp.bfloat16)
    out = jnp.dot(hid.astype(jnp.bfloat16), wd, preferred_element_type=jnp.float32)
    out = out + bd_ref[0]
    weighted = out * g_ref[0]
    for i in range(_K):
        tok = idx_sref[e, i]
        y_ref[pl.ds(tok, 1), :] += weighted[i:i + 1, :]


def _ffn_scatter(idx_ek, tokens, W_up, b_up, W_down, b_down, G_ek1):
    return pl.pallas_call(
        _ffn_body,
        grid_spec=pltpu.PrefetchScalarGridSpec(
            num_scalar_prefetch=1,
            grid=(_E,),
            in_specs=[
                pl.BlockSpec((1, _K, _H), lambda e, s: (e, 0, 0)),
                pl.BlockSpec((1, _H, _FFN), lambda e, s: (e // _GSIZE, 0, 0),
                             pipeline_mode=pl.Buffered(buffer_count=2)),
                pl.BlockSpec((1, 1, _FFN), lambda e, s: (e // _GSIZE, 0, 0)),
                pl.BlockSpec((1, _FFN, _H), lambda e, s: (e, 0, 0)),
                pl.BlockSpec((1, 1, _H), lambda e, s: (e, 0, 0)),
                pl.BlockSpec((1, _K, 1), lambda e, s: (e, 0, 0)),
            ],
            out_specs=pl.BlockSpec((_N, _H), lambda e, s: (0, 0)),
        ),
        out_shape=jax.ShapeDtypeStruct((_N, _H), jnp.float32),
    )(idx_ek, tokens, W_up, b_up, W_down, b_down, G_ek1)


def kernel(x, W_r, b_r, W_up, b_up, W_down, b_down):
    xf = x.reshape(_N, _H)
    g_ek, idx_ek = _router(xf, W_r, b_r)
    tokens = _gather(xf, idx_ek.reshape(_NW, _ROWS_PER_W))
    y = _ffn_scatter(
        idx_ek,
        tokens.reshape(_E, _K, _H),
        W_up, b_up.reshape(_GROUPS, 1, _FFN),
        W_down, b_down.reshape(_E, 1, _H),
        g_ek.reshape(_E, _K, 1),
    )
    return y.reshape(_B, _SEQ, _H)


# speedup vs baseline: 2.2329x; 1.0507x over previous
import functools

import jax
import jax.numpy as jnp
from jax import lax
from jax.experimental import pallas as pl
from jax.experimental.pallas import tpu as pltpu
from jax.experimental.pallas import tpu_sc as plsc

_B, _SEQ, _H = 2, 2048, 768
_E = 64
_GROUPS = 16
_GSIZE = _E // _GROUPS
_FFN = 2048
_N = _B * _SEQ
_K = _N // _E

_NC, _NS = 2, 16
_NW = _NC * _NS
_ROWS_PER_W = _N // _NW
_ROWS_PER_TILE = _N // _NS
_HH = _H // _NC
_CHUNK = 64
_NCHUNK = _ROWS_PER_TILE // _CHUNK


def _router_body(x_ref, wr_ref, br_ref, g_ref, i_ref):
    hT = lax.dot_general(
        wr_ref[...], x_ref[...], (((0,), (1,)), ((), ())),
        preferred_element_type=jnp.float32,
    )
    hT = hT + br_ref[...]
    m = jnp.max(hT, axis=0, keepdims=True)
    ex = jnp.exp(hT - m)
    s = jnp.sum(ex, axis=0, keepdims=True)
    sm = ex / s
    iota1 = lax.broadcasted_iota(jnp.int32, sm.shape, 1)
    iota_k = lax.broadcasted_iota(jnp.int32, (_E, _K), 1)

    def step(j, carry):
        run, g_acc, i_acc = carry
        cmax = jnp.max(run, axis=1, keepdims=True)
        isel = jnp.min(jnp.where(run == cmax, iota1, _N), axis=1, keepdims=True)
        g_acc = jnp.where(iota_k == j, cmax, g_acc)
        i_acc = jnp.where(iota_k == j, isel, i_acc)
        run = jnp.where(iota1 == isel, -1.0, run)
        return run, g_acc, i_acc

    _, g_acc, i_acc = lax.fori_loop(
        0, _K, step,
        (sm, jnp.zeros((_E, _K), jnp.float32), jnp.zeros((_E, _K), jnp.int32)),
    )
    g_ref[...] = g_acc
    i_ref[...] = i_acc


def _router(xf, W_r, b_r):
    return pl.pallas_call(
        _router_body,
        out_shape=(
            jax.ShapeDtypeStruct((_E, _K), jnp.float32),
            jax.ShapeDtypeStruct((_E, _K), jnp.int32),
        ),
    )(xf, W_r, b_r.reshape(_E, 1))


def _gather(xf, idx_w):
    mesh = plsc.VectorSubcoreMesh(core_axis_name="c", subcore_axis_name="s")

    @functools.partial(
        pl.kernel,
        mesh=mesh,
        out_type=jax.ShapeDtypeStruct((_N, _H), jnp.float32),
        scratch_types=[
            pltpu.VMEM((_ROWS_PER_W,), jnp.int32),
            pltpu.VMEM((_ROWS_PER_W, _H), jnp.float32),
            pltpu.SemaphoreType.DMA,
        ],
    )
    def k(x_hbm, idx_hbm, out_hbm, idx_v, rows_v, sem):
        wid = lax.axis_index("s") * _NC + lax.axis_index("c")
        pltpu.sync_copy(idx_hbm.at[wid], idx_v)
        pltpu.async_copy(x_hbm.at[idx_v], rows_v, sem).wait()
        pltpu.sync_copy(rows_v, out_hbm.at[pl.ds(wid * _ROWS_PER_W, _ROWS_PER_W)])

    return k(xf, idx_w)


def _ffn_body(idx_sref, t_ref, wu_hbm, bu_ref, wd_ref, bd_ref, g_ref, y_ref,
              wu_buf, sem):
    e = pl.program_id(0)
    g = e // _GSIZE

    @pl.when(e == 0)
    def _init():
        y_ref[...] = jnp.zeros((_N, _H), jnp.float32)

    def cp(gi, slot):
        return pltpu.make_async_copy(wu_hbm.at[gi], wu_buf.at[slot],
                                     sem.at[slot])

    @pl.when(e == 0)
    def _prime():
        cp(0, 0).start()

    for par in (0, 1):
        @pl.when((e % _GSIZE == 0) & (g % 2 == par))
        def _wait(par=par):
            cp(g, par).wait()

        @pl.when((e % _GSIZE == 0) & (g % 2 == par) & (g < _GROUPS - 1))
        def _issue(par=par):
            cp(g + 1, 1 - par).start()

    t = t_ref[0].astype(jnp.bfloat16)
    wu = wu_buf[g % 2].astype(jnp.bfloat16)
    up = jnp.dot(t, wu, preferred_element_type=jnp.float32)
    up = up + bu_ref[0]
    hid = up * (1.0 / (1.0 + jnp.exp(-up)))
    wd = wd_ref[0].astype(jnp.bfloat16)
    out = jnp.dot(hid.astype(jnp.bfloat16), wd, preferred_element_type=jnp.float32)
    out = out + bd_ref[0]
    weighted = out * g_ref[0]
    for i in range(_K):
        tok = idx_sref[e, i]
        y_ref[pl.ds(tok, 1), :] += weighted[i:i + 1, :]


def _ffn_scatter(idx_ek, tokens, W_up, b_up, W_down, b_down, G_ek1):
    return pl.pallas_call(
        _ffn_body,
        grid_spec=pltpu.PrefetchScalarGridSpec(
            num_scalar_prefetch=1,
            grid=(_E,),
            in_specs=[
                pl.BlockSpec((1, _K, _H), lambda e, s: (e, 0, 0)),
                pl.BlockSpec(memory_space=pl.ANY),
                pl.BlockSpec((1, 1, _FFN), lambda e, s: (e // _GSIZE, 0, 0)),
                pl.BlockSpec((1, _FFN, _H), lambda e, s: (e, 0, 0)),
                pl.BlockSpec((1, 1, _H), lambda e, s: (e, 0, 0)),
                pl.BlockSpec((1, _K, 1), lambda e, s: (e, 0, 0)),
            ],
            out_specs=pl.BlockSpec((_N, _H), lambda e, s: (0, 0)),
            scratch_shapes=[
                pltpu.VMEM((2, _H, _FFN), jnp.float32),
                pltpu.SemaphoreType.DMA((2,)),
            ],
        ),
        out_shape=jax.ShapeDtypeStruct((_N, _H), jnp.float32),
    )(idx_ek, tokens, W_up, b_up, W_down, b_down, G_ek1)


def kernel(x, W_r, b_r, W_up, b_up, W_down, b_down):
    xf = x.reshape(_N, _H)
    g_ek, idx_ek = _router(xf, W_r, b_r)
    tokens = _gather(xf, idx_ek.reshape(_NW, _ROWS_PER_W))
    y = _ffn_scatter(
        idx_ek,
        tokens.reshape(_E, _K, _H),
        W_up, b_up.reshape(_GROUPS, 1, _FFN),
        W_down, b_down.reshape(_E, 1, _H),
        g_ek.reshape(_E, _K, 1),
    )
    return y.reshape(_B, _SEQ, _H)


# speedup vs baseline: 2.2702x; 1.0167x over previous
import functools

import jax
import jax.numpy as jnp
from jax import lax
from jax.experimental import pallas as pl
from jax.experimental.pallas import tpu as pltpu
from jax.experimental.pallas import tpu_sc as plsc

_B, _SEQ, _H = 2, 2048, 768
_E = 64
_GROUPS = 16
_GSIZE = _E // _GROUPS
_FFN = 2048
_N = _B * _SEQ
_K = _N // _E

_NC, _NS = 2, 16
_NW = _NC * _NS
_ROWS_PER_W = _N // _NW
_ROWS_PER_TILE = _N // _NS
_HH = _H // _NC
_CHUNK = 64
_NCHUNK = _ROWS_PER_TILE // _CHUNK


def _router_body(x_ref, wr_ref, br_ref, g_ref, i_ref):
    hT = lax.dot_general(
        wr_ref[...], x_ref[...], (((0,), (1,)), ((), ())),
        preferred_element_type=jnp.float32,
    )
    hT = hT + br_ref[...]
    m = jnp.max(hT, axis=0, keepdims=True)
    ex = jnp.exp(hT - m)
    s = jnp.sum(ex, axis=0, keepdims=True)
    sm = ex / s
    iota1 = lax.broadcasted_iota(jnp.int32, sm.shape, 1)
    iota_k = lax.broadcasted_iota(jnp.int32, (_E, _K), 1)

    def step(j, carry):
        run, cmax, g_acc, i_acc = carry
        isel = jnp.min(jnp.where(run == cmax, iota1, _N), axis=1, keepdims=True)
        g_acc = jnp.where(iota_k == j, cmax, g_acc)
        i_acc = jnp.where(iota_k == j, isel, i_acc)
        run = jnp.where(iota1 == isel, -1.0, run)
        return run, jnp.max(run, axis=1, keepdims=True), g_acc, i_acc

    _, _, g_acc, i_acc = lax.fori_loop(
        0, _K, step,
        (sm, jnp.max(sm, axis=1, keepdims=True),
         jnp.zeros((_E, _K), jnp.float32), jnp.zeros((_E, _K), jnp.int32)),
    )
    g_ref[...] = g_acc
    i_ref[...] = i_acc


def _router(xf, W_r, b_r):
    return pl.pallas_call(
        _router_body,
        out_shape=(
            jax.ShapeDtypeStruct((_E, _K), jnp.float32),
            jax.ShapeDtypeStruct((_E, _K), jnp.int32),
        ),
    )(xf, W_r, b_r.reshape(_E, 1))


def _gather(xf, idx_w):
    mesh = plsc.VectorSubcoreMesh(core_axis_name="c", subcore_axis_name="s")

    @functools.partial(
        pl.kernel,
        mesh=mesh,
        out_type=jax.ShapeDtypeStruct((_N, _H), jnp.float32),
        scratch_types=[
            pltpu.VMEM((_ROWS_PER_W,), jnp.int32),
            pltpu.VMEM((_ROWS_PER_W, _H), jnp.float32),
            pltpu.SemaphoreType.DMA,
        ],
    )
    def k(x_hbm, idx_hbm, out_hbm, idx_v, rows_v, sem):
        wid = lax.axis_index("s") * _NC + lax.axis_index("c")
        pltpu.sync_copy(idx_hbm.at[wid], idx_v)
        pltpu.async_copy(x_hbm.at[idx_v], rows_v, sem).wait()
        pltpu.sync_copy(rows_v, out_hbm.at[pl.ds(wid * _ROWS_PER_W, _ROWS_PER_W)])

    return k(xf, idx_w)


def _ffn_body(idx_sref, t_ref, wu_hbm, bu_ref, wd_ref, bd_ref, g_ref, y_ref,
              wu_buf, sem):
    e = pl.program_id(0)
    g = e // _GSIZE

    @pl.when(e == 0)
    def _init():
        y_ref[...] = jnp.zeros((_N, _H), jnp.float32)

    def cp(gi, slot):
        return pltpu.make_async_copy(wu_hbm.at[gi], wu_buf.at[slot],
                                     sem.at[slot])

    @pl.when(e == 0)
    def _prime():
        cp(0, 0).start()

    for par in (0, 1):
        @pl.when((e % _GSIZE == 0) & (g % 2 == par))
        def _wait(par=par):
            cp(g, par).wait()

        @pl.when((e % _GSIZE == 0) & (g % 2 == par) & (g < _GROUPS - 1))
        def _issue(par=par):
            cp(g + 1, 1 - par).start()

    t = t_ref[0].astype(jnp.bfloat16)
    wu = wu_buf[g % 2].astype(jnp.bfloat16)
    up = jnp.dot(t, wu, preferred_element_type=jnp.float32)
    up = up + bu_ref[0]
    hid = up * (1.0 / (1.0 + jnp.exp(-up)))
    wd = wd_ref[0].astype(jnp.bfloat16)
    out = jnp.dot(hid.astype(jnp.bfloat16), wd, preferred_element_type=jnp.float32)
    out = out + bd_ref[0]
    weighted = out * g_ref[0]
    for i in range(_K):
        tok = idx_sref[e, i]
        y_ref[pl.ds(tok, 1), :] += weighted[i:i + 1, :]


def _ffn_scatter(idx_ek, tokens, W_up, b_up, W_down, b_down, G_ek1):
    return pl.pallas_call(
        _ffn_body,
        grid_spec=pltpu.PrefetchScalarGridSpec(
            num_scalar_prefetch=1,
            grid=(_E,),
            in_specs=[
                pl.BlockSpec((1, _K, _H), lambda e, s: (e, 0, 0)),
                pl.BlockSpec(memory_space=pl.ANY),
                pl.BlockSpec((1, 1, _FFN), lambda e, s: (e // _GSIZE, 0, 0)),
                pl.BlockSpec((1, _FFN, _H), lambda e, s: (e, 0, 0)),
                pl.BlockSpec((1, 1, _H), lambda e, s: (e, 0, 0)),
                pl.BlockSpec((1, _K, 1), lambda e, s: (e, 0, 0)),
            ],
            out_specs=pl.BlockSpec((_N, _H), lambda e, s: (0, 0)),
            scratch_shapes=[
                pltpu.VMEM((2, _H, _FFN), jnp.float32),
                pltpu.SemaphoreType.DMA((2,)),
            ],
        ),
        out_shape=jax.ShapeDtypeStruct((_N, _H), jnp.float32),
    )(idx_ek, tokens, W_up, b_up, W_down, b_down, G_ek1)


def kernel(x, W_r, b_r, W_up, b_up, W_down, b_down):
    xf = x.reshape(_N, _H)
    g_ek, idx_ek = _router(xf, W_r, b_r)
    tokens = _gather(xf, idx_ek.reshape(_NW, _ROWS_PER_W))
    y = _ffn_scatter(
        idx_ek,
        tokens.reshape(_E, _K, _H),
        W_up, b_up.reshape(_GROUPS, 1, _FFN),
        W_down, b_down.reshape(_E, 1, _H),
        g_ek.reshape(_E, _K, 1),
    )
    return y.reshape(_B, _SEQ, _H)


# speedup vs baseline: 2.3951x; 1.0551x over previous
import functools

import jax
import jax.numpy as jnp
from jax import lax
from jax.experimental import pallas as pl
from jax.experimental.pallas import tpu as pltpu
from jax.experimental.pallas import tpu_sc as plsc

_B, _SEQ, _H = 2, 2048, 768
_E = 64
_GROUPS = 16
_GSIZE = _E // _GROUPS
_FFN = 2048
_N = _B * _SEQ
_K = _N // _E

_NC, _NS = 2, 16
_NW = _NC * _NS
_ROWS_PER_W = _N // _NW
_ROWS_PER_TILE = _N // _NS
_HH = _H // _NC
_CHUNK = 64
_NCHUNK = _ROWS_PER_TILE // _CHUNK


def _router_body(x_ref, wr_ref, br_ref, g_ref, i_ref):
    hT = lax.dot_general(
        wr_ref[...], x_ref[...], (((0,), (1,)), ((), ())),
        preferred_element_type=jnp.float32,
    )
    hT = hT + br_ref[...]
    m = jnp.max(hT, axis=0, keepdims=True)
    ex = jnp.exp(hT - m)
    s = jnp.sum(ex, axis=0, keepdims=True)
    sm = ex / s
    iota1 = lax.broadcasted_iota(jnp.int32, sm.shape, 1)
    iota_k = lax.broadcasted_iota(jnp.int32, (_E, _K), 1)

    def step(j, carry):
        run, cmax, g_acc, i_acc = carry
        isel = jnp.min(jnp.where(run == cmax, iota1, _N), axis=1, keepdims=True)
        g_acc = jnp.where(iota_k == j, cmax, g_acc)
        i_acc = jnp.where(iota_k == j, isel, i_acc)
        run = jnp.where(iota1 == isel, -1.0, run)
        return run, jnp.max(run, axis=1, keepdims=True), g_acc, i_acc

    _, _, g_acc, i_acc = lax.fori_loop(
        0, _K, step,
        (sm, jnp.max(sm, axis=1, keepdims=True),
         jnp.zeros((_E, _K), jnp.float32), jnp.zeros((_E, _K), jnp.int32)),
        unroll=4,
    )
    g_ref[...] = g_acc
    i_ref[...] = i_acc


def _router(xf, W_r, b_r):
    return pl.pallas_call(
        _router_body,
        out_shape=(
            jax.ShapeDtypeStruct((_E, _K), jnp.float32),
            jax.ShapeDtypeStruct((_E, _K), jnp.int32),
        ),
    )(xf, W_r, b_r.reshape(_E, 1))


def _gather(xf, idx_w):
    mesh = plsc.VectorSubcoreMesh(core_axis_name="c", subcore_axis_name="s")

    @functools.partial(
        pl.kernel,
        mesh=mesh,
        out_type=jax.ShapeDtypeStruct((_N, _H), jnp.float32),
        scratch_types=[
            pltpu.VMEM((_ROWS_PER_W,), jnp.int32),
            pltpu.VMEM((_ROWS_PER_W, _H), jnp.float32),
            pltpu.SemaphoreType.DMA,
        ],
    )
    def k(x_hbm, idx_hbm, out_hbm, idx_v, rows_v, sem):
        wid = lax.axis_index("s") * _NC + lax.axis_index("c")
        pltpu.sync_copy(idx_hbm.at[wid], idx_v)
        pltpu.async_copy(x_hbm.at[idx_v], rows_v, sem).wait()
        pltpu.sync_copy(rows_v, out_hbm.at[pl.ds(wid * _ROWS_PER_W, _ROWS_PER_W)])

    return k(xf, idx_w)


def _ffn_body(idx_sref, t_ref, wu_hbm, bu_ref, wd_ref, bd_ref, g_ref, y_ref,
              wu_buf, sem):
    e = pl.program_id(0)
    g = e // _GSIZE

    @pl.when(e == 0)
    def _init():
        y_ref[...] = jnp.zeros((_N, _H), jnp.float32)

    def cp(gi, slot):
        return pltpu.make_async_copy(wu_hbm.at[gi], wu_buf.at[slot],
                                     sem.at[slot])

    @pl.when(e == 0)
    def _prime():
        cp(0, 0).start()

    for par in (0, 1):
        @pl.when((e % _GSIZE == 0) & (g % 2 == par))
        def _wait(par=par):
            cp(g, par).wait()

        @pl.when((e % _GSIZE == 0) & (g % 2 == par) & (g < _GROUPS - 1))
        def _issue(par=par):
            cp(g + 1, 1 - par).start()

    t = t_ref[0].astype(jnp.bfloat16)
    wu = wu_buf[g % 2].astype(jnp.bfloat16)
    up = jnp.dot(t, wu, preferred_element_type=jnp.float32)
    up = up + bu_ref[0]
    hid = up * (1.0 / (1.0 + jnp.exp(-up)))
    wd = wd_ref[0].astype(jnp.bfloat16)
    out = jnp.dot(hid.astype(jnp.bfloat16), wd, preferred_element_type=jnp.float32)
    out = out + bd_ref[0]
    weighted = out * g_ref[0]
    for i in range(_K):
        tok = idx_sref[e, i]
        y_ref[pl.ds(tok, 1), :] += weighted[i:i + 1, :]


def _ffn_scatter(idx_ek, tokens, W_up, b_up, W_down, b_down, G_ek1):
    return pl.pallas_call(
        _ffn_body,
        grid_spec=pltpu.PrefetchScalarGridSpec(
            num_scalar_prefetch=1,
            grid=(_E,),
            in_specs=[
                pl.BlockSpec((1, _K, _H), lambda e, s: (e, 0, 0)),
                pl.BlockSpec(memory_space=pl.ANY),
                pl.BlockSpec((1, 1, _FFN), lambda e, s: (e // _GSIZE, 0, 0)),
                pl.BlockSpec((1, _FFN, _H), lambda e, s: (e, 0, 0)),
                pl.BlockSpec((1, 1, _H), lambda e, s: (e, 0, 0)),
                pl.BlockSpec((1, _K, 1), lambda e, s: (e, 0, 0)),
            ],
            out_specs=pl.BlockSpec((_N, _H), lambda e, s: (0, 0)),
            scratch_shapes=[
                pltpu.VMEM((2, _H, _FFN), jnp.float32),
                pltpu.SemaphoreType.DMA((2,)),
            ],
        ),
        out_shape=jax.ShapeDtypeStruct((_N, _H), jnp.float32),
    )(idx_ek, tokens, W_up, b_up, W_down, b_down, G_ek1)


def kernel(x, W_r, b_r, W_up, b_up, W_down, b_down):
    xf = x.reshape(_N, _H)
    g_ek, idx_ek = _router(xf, W_r, b_r)
    tokens = _gather(xf, idx_ek.reshape(_NW, _ROWS_PER_W))
    y = _ffn_scatter(
        idx_ek,
        tokens.reshape(_E, _K, _H),
        W_up, b_up.reshape(_GROUPS, 1, _FFN),
        W_down, b_down.reshape(_E, 1, _H),
        g_ek.reshape(_E, _K, 1),
    )
    return y.reshape(_B, _SEQ, _H)


# speedup vs baseline: 2.4176x; 1.0094x over previous
import functools

import jax
import jax.numpy as jnp
from jax import lax
from jax.experimental import pallas as pl
from jax.experimental.pallas import tpu as pltpu
from jax.experimental.pallas import tpu_sc as plsc

_B, _SEQ, _H = 2, 2048, 768
_E = 64
_GROUPS = 16
_GSIZE = _E // _GROUPS
_FFN = 2048
_N = _B * _SEQ
_K = _N // _E

_NC, _NS = 2, 16
_NW = _NC * _NS
_ROWS_PER_W = _N // _NW
_ROWS_PER_TILE = _N // _NS
_HH = _H // _NC
_CHUNK = 64
_NCHUNK = _ROWS_PER_TILE // _CHUNK


def _router_body(x_ref, wr_ref, br_ref, g_ref, i_ref):
    hT = lax.dot_general(
        wr_ref[...], x_ref[...], (((0,), (1,)), ((), ())),
        preferred_element_type=jnp.float32,
    )
    hT = hT + br_ref[...]
    m = jnp.max(hT, axis=0, keepdims=True)
    ex = jnp.exp(hT - m)
    s = jnp.sum(ex, axis=0, keepdims=True)
    sm = ex / s
    iota1 = lax.broadcasted_iota(jnp.int32, sm.shape, 1)
    iota_k = lax.broadcasted_iota(jnp.int32, (_E, _K), 1)

    def step(j, carry):
        run, cmax, g_acc, i_acc = carry
        isel = jnp.min(jnp.where(run == cmax, iota1, _N), axis=1, keepdims=True)
        g_acc = jnp.where(iota_k == j, cmax, g_acc)
        i_acc = jnp.where(iota_k == j, isel, i_acc)
        run = jnp.where(iota1 == isel, -1.0, run)
        return run, jnp.max(run, axis=1, keepdims=True), g_acc, i_acc

    _, _, g_acc, i_acc = lax.fori_loop(
        0, _K, step,
        (sm, jnp.max(sm, axis=1, keepdims=True),
         jnp.zeros((_E, _K), jnp.float32), jnp.zeros((_E, _K), jnp.int32)),
        unroll=8,
    )
    g_ref[...] = g_acc
    i_ref[...] = i_acc


def _router(xf, W_r, b_r):
    return pl.pallas_call(
        _router_body,
        out_shape=(
            jax.ShapeDtypeStruct((_E, _K), jnp.float32),
            jax.ShapeDtypeStruct((_E, _K), jnp.int32),
        ),
    )(xf, W_r, b_r.reshape(_E, 1))


def _gather(xf, idx_w):
    mesh = plsc.VectorSubcoreMesh(core_axis_name="c", subcore_axis_name="s")

    @functools.partial(
        pl.kernel,
        mesh=mesh,
        out_type=jax.ShapeDtypeStruct((_N, _H), jnp.float32),
        scratch_types=[
            pltpu.VMEM((_ROWS_PER_W,), jnp.int32),
            pltpu.VMEM((_ROWS_PER_W, _H), jnp.float32),
            pltpu.SemaphoreType.DMA,
        ],
    )
    def k(x_hbm, idx_hbm, out_hbm, idx_v, rows_v, sem):
        wid = lax.axis_index("s") * _NC + lax.axis_index("c")
        pltpu.sync_copy(idx_hbm.at[wid], idx_v)
        pltpu.async_copy(x_hbm.at[idx_v], rows_v, sem).wait()
        pltpu.sync_copy(rows_v, out_hbm.at[pl.ds(wid * _ROWS_PER_W, _ROWS_PER_W)])

    return k(xf, idx_w)


def _ffn_body(idx_sref, t_ref, wu_hbm, bu_ref, wd_ref, bd_ref, g_ref, y_ref,
              wu_buf, sem):
    e = pl.program_id(0)
    g = e // _GSIZE

    @pl.when(e == 0)
    def _init():
        y_ref[...] = jnp.zeros((_N, _H), jnp.float32)

    def cp(gi, slot):
        return pltpu.make_async_copy(wu_hbm.at[gi], wu_buf.at[slot],
                                     sem.at[slot])

    @pl.when(e == 0)
    def _prime():
        cp(0, 0).start()

    for par in (0, 1):
        @pl.when((e % _GSIZE == 0) & (g % 2 == par))
        def _wait(par=par):
            cp(g, par).wait()

        @pl.when((e % _GSIZE == 0) & (g % 2 == par) & (g < _GROUPS - 1))
        def _issue(par=par):
            cp(g + 1, 1 - par).start()

    t = t_ref[0].astype(jnp.bfloat16)
    wu = wu_buf[g % 2].astype(jnp.bfloat16)
    up = jnp.dot(t, wu, preferred_element_type=jnp.float32)
    up = up + bu_ref[0]
    hid = up * (1.0 / (1.0 + jnp.exp(-up)))
    wd = wd_ref[0].astype(jnp.bfloat16)
    out = jnp.dot(hid.astype(jnp.bfloat16), wd, preferred_element_type=jnp.float32)
    out = out + bd_ref[0]
    weighted = out * g_ref[0]
    for i in range(_K):
        tok = idx_sref[e, i]
        y_ref[pl.ds(tok, 1), :] += weighted[i:i + 1, :]


def _ffn_scatter(idx_ek, tokens, W_up, b_up, W_down, b_down, G_ek1):
    return pl.pallas_call(
        _ffn_body,
        grid_spec=pltpu.PrefetchScalarGridSpec(
            num_scalar_prefetch=1,
            grid=(_E,),
            in_specs=[
                pl.BlockSpec((1, _K, _H), lambda e, s: (e, 0, 0)),
                pl.BlockSpec(memory_space=pl.ANY),
                pl.BlockSpec((1, 1, _FFN), lambda e, s: (e // _GSIZE, 0, 0)),
                pl.BlockSpec((1, _FFN, _H), lambda e, s: (e, 0, 0)),
                pl.BlockSpec((1, 1, _H), lambda e, s: (e, 0, 0)),
                pl.BlockSpec((1, _K, 1), lambda e, s: (e, 0, 0)),
            ],
            out_specs=pl.BlockSpec((_N, _H), lambda e, s: (0, 0)),
            scratch_shapes=[
                pltpu.VMEM((2, _H, _FFN), jnp.float32),
                pltpu.SemaphoreType.DMA((2,)),
            ],
        ),
        out_shape=jax.ShapeDtypeStruct((_N, _H), jnp.float32),
    )(idx_ek, tokens, W_up, b_up, W_down, b_down, G_ek1)


def kernel(x, W_r, b_r, W_up, b_up, W_down, b_down):
    xf = x.reshape(_N, _H)
    g_ek, idx_ek = _router(xf, W_r, b_r)
    tokens = _gather(xf, idx_ek.reshape(_NW, _ROWS_PER_W))
    y = _ffn_scatter(
        idx_ek,
        tokens.reshape(_E, _K, _H),
        W_up, b_up.reshape(_GROUPS, 1, _FFN),
        W_down, b_down.reshape(_E, 1, _H),
        g_ek.reshape(_E, _K, 1),
    )
    return y.reshape(_B, _SEQ, _H)


# speedup vs baseline: 2.4185x; 1.0004x over previous
import functools

import jax
import jax.numpy as jnp
from jax import lax
from jax.experimental import pallas as pl
from jax.experimental.pallas import tpu as pltpu
from jax.experimental.pallas import tpu_sc as plsc

_B, _SEQ, _H = 2, 2048, 768
_E = 64
_GROUPS = 16
_GSIZE = _E // _GROUPS
_FFN = 2048
_N = _B * _SEQ
_K = _N // _E

_NC, _NS = 2, 16
_NW = _NC * _NS
_ROWS_PER_W = _N // _NW
_ROWS_PER_TILE = _N // _NS
_HH = _H // _NC
_CHUNK = 64
_NCHUNK = _ROWS_PER_TILE // _CHUNK


def _router_body(x_ref, wr_ref, br_ref, g_ref, i_ref):
    hT = lax.dot_general(
        wr_ref[...], x_ref[...], (((0,), (1,)), ((), ())),
        preferred_element_type=jnp.float32,
    )
    hT = hT + br_ref[...]
    m = jnp.max(hT, axis=0, keepdims=True)
    ex = jnp.exp(hT - m)
    s = jnp.sum(ex, axis=0, keepdims=True)
    sm = ex / s
    iota1 = lax.broadcasted_iota(jnp.int32, sm.shape, 1)
    iota_k = lax.broadcasted_iota(jnp.int32, (_E, _K), 1)

    def step(j, carry):
        run, cmax, g_acc, i_acc = carry
        isel = jnp.min(jnp.where(run == cmax, iota1, _N), axis=1, keepdims=True)
        g_acc = jnp.where(iota_k == j, cmax, g_acc)
        i_acc = jnp.where(iota_k == j, isel, i_acc)
        run = jnp.where(iota1 == isel, -1.0, run)
        return run, jnp.max(run, axis=1, keepdims=True), g_acc, i_acc

    _, _, g_acc, i_acc = lax.fori_loop(
        0, _K, step,
        (sm, jnp.max(sm, axis=1, keepdims=True),
         jnp.zeros((_E, _K), jnp.float32), jnp.zeros((_E, _K), jnp.int32)),
        unroll=16,
    )
    g_ref[...] = g_acc
    i_ref[...] = i_acc


def _router(xf, W_r, b_r):
    return pl.pallas_call(
        _router_body,
        out_shape=(
            jax.ShapeDtypeStruct((_E, _K), jnp.float32),
            jax.ShapeDtypeStruct((_E, _K), jnp.int32),
        ),
    )(xf, W_r, b_r.reshape(_E, 1))


def _gather(xf, idx_w):
    mesh = plsc.VectorSubcoreMesh(core_axis_name="c", subcore_axis_name="s")

    @functools.partial(
        pl.kernel,
        mesh=mesh,
        out_type=jax.ShapeDtypeStruct((_N, _H), jnp.float32),
        scratch_types=[
            pltpu.VMEM((_ROWS_PER_W,), jnp.int32),
            pltpu.VMEM((_ROWS_PER_W, _H), jnp.float32),
            pltpu.SemaphoreType.DMA,
        ],
    )
    def k(x_hbm, idx_hbm, out_hbm, idx_v, rows_v, sem):
        wid = lax.axis_index("s") * _NC + lax.axis_index("c")
        pltpu.sync_copy(idx_hbm.at[wid], idx_v)
        pltpu.async_copy(x_hbm.at[idx_v], rows_v, sem).wait()
        pltpu.sync_copy(rows_v, out_hbm.at[pl.ds(wid * _ROWS_PER_W, _ROWS_PER_W)])

    return k(xf, idx_w)


def _ffn_body(idx_sref, t_ref, wu_hbm, bu_ref, wd_ref, bd_ref, g_ref, y_ref,
              wu_buf, sem):
    e = pl.program_id(0)
    g = e // _GSIZE

    @pl.when(e == 0)
    def _init():
        y_ref[...] = jnp.zeros((_N, _H), jnp.float32)

    def cp(gi, slot):
        return pltpu.make_async_copy(wu_hbm.at[gi], wu_buf.at[slot],
                                     sem.at[slot])

    @pl.when(e == 0)
    def _prime():
        cp(0, 0).start()

    for par in (0, 1):
        @pl.when((e % _GSIZE == 0) & (g % 2 == par))
        def _wait(par=par):
            cp(g, par).wait()

        @pl.when((e % _GSIZE == 0) & (g % 2 == par) & (g < _GROUPS - 1))
        def _issue(par=par):
            cp(g + 1, 1 - par).start()

    t = t_ref[0].astype(jnp.bfloat16)
    wu = wu_buf[g % 2].astype(jnp.bfloat16)
    up = jnp.dot(t, wu, preferred_element_type=jnp.float32)
    up = up + bu_ref[0]
    hid = up * (1.0 / (1.0 + jnp.exp(-up)))
    wd = wd_ref[0].astype(jnp.bfloat16)
    out = jnp.dot(hid.astype(jnp.bfloat16), wd, preferred_element_type=jnp.float32)
    out = out + bd_ref[0]
    weighted = out * g_ref[0]
    for i in range(_K):
        tok = idx_sref[e, i]
        y_ref[pl.ds(tok, 1), :] += weighted[i:i + 1, :]


def _ffn_scatter(idx_ek, tokens, W_up, b_up, W_down, b_down, G_ek1):
    return pl.pallas_call(
        _ffn_body,
        grid_spec=pltpu.PrefetchScalarGridSpec(
            num_scalar_prefetch=1,
            grid=(_E,),
            in_specs=[
                pl.BlockSpec((1, _K, _H), lambda e, s: (e, 0, 0)),
                pl.BlockSpec(memory_space=pl.ANY),
                pl.BlockSpec((1, 1, _FFN), lambda e, s: (e // _GSIZE, 0, 0)),
                pl.BlockSpec((1, _FFN, _H), lambda e, s: (e, 0, 0)),
                pl.BlockSpec((1, 1, _H), lambda e, s: (e, 0, 0)),
                pl.BlockSpec((1, _K, 1), lambda e, s: (e, 0, 0)),
            ],
            out_specs=pl.BlockSpec((_N, _H), lambda e, s: (0, 0)),
            scratch_shapes=[
                pltpu.VMEM((2, _H, _FFN), jnp.float32),
                pltpu.SemaphoreType.DMA((2,)),
            ],
        ),
        out_shape=jax.ShapeDtypeStruct((_N, _H), jnp.float32),
    )(idx_ek, tokens, W_up, b_up, W_down, b_down, G_ek1)


def kernel(x, W_r, b_r, W_up, b_up, W_down, b_down):
    xf = x.reshape(_N, _H)
    g_ek, idx_ek = _router(xf, W_r, b_r)
    tokens = _gather(xf, idx_ek.reshape(_NW, _ROWS_PER_W))
    y = _ffn_scatter(
        idx_ek,
        tokens.reshape(_E, _K, _H),
        W_up, b_up.reshape(_GROUPS, 1, _FFN),
        W_down, b_down.reshape(_E, 1, _H),
        g_ek.reshape(_E, _K, 1),
    )
    return y.reshape(_B, _SEQ, _H)


# speedup vs baseline: 2.6131x; 1.0805x over previous
import functools

import jax
import jax.numpy as jnp
from jax import lax
from jax.experimental import pallas as pl
from jax.experimental.pallas import tpu as pltpu
from jax.experimental.pallas import tpu_sc as plsc

_B, _SEQ, _H = 2, 2048, 768
_E = 64
_GROUPS = 16
_GSIZE = _E // _GROUPS
_FFN = 2048
_N = _B * _SEQ
_K = _N // _E

_NC, _NS = 2, 16
_NW = _NC * _NS
_ROWS_PER_W = _N // _NW
_ROWS_PER_TILE = _N // _NS
_HH = _H // _NC
_CHUNK = 64
_NCHUNK = _ROWS_PER_TILE // _CHUNK


def _router_body(x_ref, wr_ref, br_ref, g_ref, i_ref):
    hT = lax.dot_general(
        wr_ref[...], x_ref[...], (((0,), (1,)), ((), ())),
        preferred_element_type=jnp.float32,
    )
    hT = hT + br_ref[...]
    m = jnp.max(hT, axis=0, keepdims=True)
    ex = jnp.exp(hT - m)
    s = jnp.sum(ex, axis=0, keepdims=True)
    sm = ex / s
    iota1 = lax.broadcasted_iota(jnp.int32, sm.shape, 1)
    iota_k = lax.broadcasted_iota(jnp.int32, (_E, _K), 1)

    def step(j, carry):
        run, cmax, g_acc, i_acc = carry
        isel = jnp.min(jnp.where(run == cmax, iota1, _N), axis=1, keepdims=True)
        g_acc = jnp.where(iota_k == j, cmax, g_acc)
        i_acc = jnp.where(iota_k == j, isel, i_acc)
        run = jnp.where(iota1 == isel, -1.0, run)
        return run, jnp.max(run, axis=1, keepdims=True), g_acc, i_acc

    _, _, g_acc, i_acc = lax.fori_loop(
        0, _K, step,
        (sm, jnp.max(sm, axis=1, keepdims=True),
         jnp.zeros((_E, _K), jnp.float32), jnp.zeros((_E, _K), jnp.int32)),
        unroll=8,
    )
    g_ref[...] = g_acc
    i_ref[...] = i_acc


def _router(xf, W_r, b_r):
    return pl.pallas_call(
        _router_body,
        out_shape=(
            jax.ShapeDtypeStruct((_E, _K), jnp.float32),
            jax.ShapeDtypeStruct((_E, _K), jnp.int32),
        ),
    )(xf, W_r, b_r.reshape(_E, 1))


def _gather(xf, idx_w):
    mesh = plsc.VectorSubcoreMesh(core_axis_name="c", subcore_axis_name="s")

    @functools.partial(
        pl.kernel,
        mesh=mesh,
        out_type=jax.ShapeDtypeStruct((_N, _H), jnp.float32),
        scratch_types=[
            pltpu.VMEM((_ROWS_PER_W,), jnp.int32),
            pltpu.VMEM((_ROWS_PER_W, _H), jnp.float32),
            pltpu.SemaphoreType.DMA,
        ],
    )
    def k(x_hbm, idx_hbm, out_hbm, idx_v, rows_v, sem):
        wid = lax.axis_index("s") * _NC + lax.axis_index("c")
        pltpu.sync_copy(idx_hbm.at[wid], idx_v)
        pltpu.async_copy(x_hbm.at[idx_v], rows_v, sem).wait()
        pltpu.sync_copy(rows_v, out_hbm.at[pl.ds(wid * _ROWS_PER_W, _ROWS_PER_W)])

    return k(xf, idx_w)


def _ffn_body(idx_sref, t_ref, wu_hbm, bu_ref, wd_hbm, bd_ref, g_ref, y_ref,
              wu_buf, wd_buf, usem, dsem):
    e = pl.program_id(0)
    g = e // _GSIZE

    @pl.when(e == 0)
    def _init():
        y_ref[...] = jnp.zeros((_N, _H), jnp.float32)

    def ucp(gi, slot):
        return pltpu.make_async_copy(wu_hbm.at[gi], wu_buf.at[slot],
                                     usem.at[slot])

    def dcp(ei, slot):
        return pltpu.make_async_copy(wd_hbm.at[ei], wd_buf.at[slot],
                                     dsem.at[slot])

    @pl.when(e == 0)
    def _prime():
        ucp(0, 0).start()
        dcp(0, 0).start()
        dcp(1, 1).start()
        dcp(2, 2).start()

    for par in (0, 1):
        @pl.when((e % _GSIZE == 0) & (g % 2 == par))
        def _uwait(par=par):
            ucp(g, par).wait()

        @pl.when((e % _GSIZE == 0) & (g % 2 == par) & (g < _GROUPS - 1))
        def _uissue(par=par):
            ucp(g + 1, 1 - par).start()

    for ph in (0, 1, 2):
        @pl.when(e % 3 == ph)
        def _dwait(ph=ph):
            dcp(e, ph).wait()

        @pl.when((e % 3 == ph) & (e < _E - 3))
        def _dissue(ph=ph):
            dcp(e + 3, ph).start()

    t = t_ref[0].astype(jnp.bfloat16)
    wu = wu_buf[g % 2].astype(jnp.bfloat16)
    up = jnp.dot(t, wu, preferred_element_type=jnp.float32)
    up = up + bu_ref[0]
    hid = up * (1.0 / (1.0 + jnp.exp(-up)))
    wd = wd_buf[e % 3].astype(jnp.bfloat16)
    out = jnp.dot(hid.astype(jnp.bfloat16), wd, preferred_element_type=jnp.float32)
    out = out + bd_ref[0]
    weighted = out * g_ref[0]
    for i in range(_K):
        tok = idx_sref[e, i]
        y_ref[pl.ds(tok, 1), :] += weighted[i:i + 1, :]


def _ffn_scatter(idx_ek, tokens, W_up, b_up, W_down, b_down, G_ek1):
    return pl.pallas_call(
        _ffn_body,
        grid_spec=pltpu.PrefetchScalarGridSpec(
            num_scalar_prefetch=1,
            grid=(_E,),
            in_specs=[
                pl.BlockSpec((1, _K, _H), lambda e, s: (e, 0, 0)),
                pl.BlockSpec(memory_space=pl.ANY),
                pl.BlockSpec((1, 1, _FFN), lambda e, s: (e // _GSIZE, 0, 0)),
                pl.BlockSpec(memory_space=pl.ANY),
                pl.BlockSpec((1, 1, _H), lambda e, s: (e, 0, 0)),
                pl.BlockSpec((1, _K, 1), lambda e, s: (e, 0, 0)),
            ],
            out_specs=pl.BlockSpec((_N, _H), lambda e, s: (0, 0)),
            scratch_shapes=[
                pltpu.VMEM((2, _H, _FFN), jnp.float32),
                pltpu.VMEM((3, _FFN, _H), jnp.float32),
                pltpu.SemaphoreType.DMA((2,)),
                pltpu.SemaphoreType.DMA((3,)),
            ],
        ),
        out_shape=jax.ShapeDtypeStruct((_N, _H), jnp.float32),
    )(idx_ek, tokens, W_up, b_up, W_down, b_down, G_ek1)


def kernel(x, W_r, b_r, W_up, b_up, W_down, b_down):
    xf = x.reshape(_N, _H)
    g_ek, idx_ek = _router(xf, W_r, b_r)
    tokens = _gather(xf, idx_ek.reshape(_NW, _ROWS_PER_W))
    y = _ffn_scatter(
        idx_ek,
        tokens.reshape(_E, _K, _H),
        W_up, b_up.reshape(_GROUPS, 1, _FFN),
        W_down, b_down.reshape(_E, 1, _H),
        g_ek.reshape(_E, _K, 1),
    )
    return y.reshape(_B, _SEQ, _H)
